# Initial kernel scaffold; baseline (speedup 1.0000x reference)
import jax
import jax.numpy as jnp
from jax.experimental import pallas as pl


def kernel(x, table, W1, b1, W2, b2):
    raise NotImplementedError("write your pallas kernel here")


# speedup vs baseline: 1.0819x; 1.0819x over previous
import functools

import jax
import jax.numpy as jnp
from jax import lax
from jax.experimental import pallas as pl
from jax.experimental.pallas import tpu as pltpu
from jax.experimental.pallas import tpu_sc as plsc

NC = 2
NS = 16
NW = NC * NS
CH = 128


def _sc_gather(table, idx3, n_rows, d):
    n_chunks = idx3.shape[1]
    b_per_w = n_chunks * CH
    mesh = plsc.VectorSubcoreMesh(core_axis_name="c", subcore_axis_name="s")

    @functools.partial(
        pl.kernel,
        out_type=jax.ShapeDtypeStruct((n_rows, d), jnp.float32),
        mesh=mesh,
        scratch_types=[
            pltpu.VMEM((n_chunks, CH), jnp.int32),
            pltpu.VMEM((CH, d), jnp.float32),
            pltpu.SemaphoreType.DMA,
        ],
        compiler_params=pltpu.CompilerParams(use_tc_tiling_on_sc=False),
    )
    def gather_kernel(table_hbm, idx_hbm, out_hbm, idx_v, rows_v, sem):
        wid = lax.axis_index("s") * NC + lax.axis_index("c")
        pltpu.sync_copy(idx_hbm.at[wid], idx_v)
        base = wid * b_per_w

        def body(c, carry):
            pltpu.async_copy(table_hbm.at[idx_v.at[c]], rows_v, sem).wait()
            pltpu.sync_copy(rows_v, out_hbm.at[pl.ds(base + c * CH, CH)])
            return carry

        lax.fori_loop(0, n_chunks, body, 0)

    return gather_kernel(table, idx3)


def _mlp_body(h_ref, w1t_ref, b1_ref, w2t_ref, b2_ref, o_ref):
    h = h_ref[...]
    a = jnp.dot(h, w1t_ref[...], preferred_element_type=jnp.float32)
    a = jnp.maximum(a + b1_ref[...], 0.0)
    o = jnp.dot(a, w2t_ref[...], preferred_element_type=jnp.float32)
    o_ref[...] = jnp.maximum(o + b2_ref[...], 0.0)


def _tc_mlp(g, w1t, b1, w2t, b2, block_rows=4096):
    n_rows, d = g.shape
    grid = (n_rows // block_rows,)
    return pl.pallas_call(
        _mlp_body,
        grid=grid,
        in_specs=[
            pl.BlockSpec((block_rows, d), lambda i: (i, 0)),
            pl.BlockSpec((d, d), lambda i: (0, 0)),
            pl.BlockSpec((1, d), lambda i: (0, 0)),
            pl.BlockSpec((d, d), lambda i: (0, 0)),
            pl.BlockSpec((1, d), lambda i: (0, 0)),
        ],
        out_specs=pl.BlockSpec((block_rows, d), lambda i: (i, 0)),
        out_shape=jax.ShapeDtypeStruct((n_rows, d), jnp.float32),
    )(g, w1t, b1, w2t, b2)


def kernel(x, table, W1, b1, W2, b2):
    B, L = x.shape
    V, D = table.shape
    n_rows = B * L
    idx3 = x.reshape(NW, n_rows // (NW * CH), CH).astype(jnp.int32)
    gathered = _sc_gather(table, idx3, n_rows, D)
    out = _tc_mlp(gathered, W1.T, b1.reshape(1, D), W2.T, b2.reshape(1, D))
    return out.reshape(B, L, D)


# speedup vs baseline: 1.2532x; 1.1583x over previous
import functools

import jax
import jax.numpy as jnp
from jax import lax
from jax.experimental import pallas as pl
from jax.experimental.pallas import tpu as pltpu
from jax.experimental.pallas import tpu_sc as plsc

NC = 2
NS = 16
NW = NC * NS
CH = 128


def _sc_gather(table, idx3, n_rows, d):
    n_chunks = idx3.shape[1]
    b_per_w = n_chunks * CH
    mesh = plsc.VectorSubcoreMesh(core_axis_name="c", subcore_axis_name="s")

    @functools.partial(
        pl.kernel,
        out_type=jax.ShapeDtypeStruct((n_rows // 2, 2 * d), jnp.float32),
        mesh=mesh,
        scratch_types=[
            pltpu.VMEM((n_chunks, CH), jnp.int32),
            pltpu.VMEM((CH, d), jnp.float32),
            pltpu.SemaphoreType.DMA,
        ],
        compiler_params=pltpu.CompilerParams(use_tc_tiling_on_sc=False),
    )
    def gather_kernel(table_hbm, idx_hbm, out_hbm, idx_v, rows_v, sem):
        wid = lax.axis_index("s") * NC + lax.axis_index("c")
        pltpu.sync_copy(idx_hbm.at[wid], idx_v)
        base = wid * b_per_w

        def body(c, carry):
            pltpu.async_copy(table_hbm.at[idx_v.at[c]], rows_v, sem).wait()
            prow = (base + c * CH) // 2
            pltpu.sync_copy(
                rows_v.at[pl.ds(0, CH // 2)],
                out_hbm.at[pl.ds(prow, CH // 2), pl.ds(0, d)],
            )
            pltpu.sync_copy(
                rows_v.at[pl.ds(CH // 2, CH // 2)],
                out_hbm.at[pl.ds(prow, CH // 2), pl.ds(d, d)],
            )
            return carry

        lax.fori_loop(0, n_chunks, body, 0)

    return gather_kernel(table, idx3)


def _mlp_body(n_b, n_l, d, h_ref, w1_ref, b1_ref, w2_ref, b2_ref, o_ref):
    h = h_ref[...]
    a = jnp.dot(h, w1_ref[...], preferred_element_type=jnp.float32)
    a = jnp.maximum(a + b1_ref[...], 0.0)
    o = jnp.dot(a, w2_ref[...], preferred_element_type=jnp.float32)
    o = jnp.maximum(o + b2_ref[...], 0.0)
    e = o[:, 0:d]
    q = o[:, d:2 * d]
    s = jnp.stack([e, q], axis=1)
    o_ref[...] = s.reshape(n_b, n_l, d)


def _tc_mlp(g, w1d, b1d, w2d, b2d, n_batch, n_hist, d, block_b=128):
    n_blocks = n_batch // block_b
    block_rows = block_b * n_hist // 2
    body = functools.partial(_mlp_body, block_b, n_hist, d)
    return pl.pallas_call(
        body,
        grid=(n_blocks,),
        in_specs=[
            pl.BlockSpec((block_rows, 2 * d), lambda i: (i, 0)),
            pl.BlockSpec((2 * d, 2 * d), lambda i: (0, 0)),
            pl.BlockSpec((1, 2 * d), lambda i: (0, 0)),
            pl.BlockSpec((2 * d, 2 * d), lambda i: (0, 0)),
            pl.BlockSpec((1, 2 * d), lambda i: (0, 0)),
        ],
        out_specs=pl.BlockSpec((block_b, n_hist, d), lambda i: (i, 0, 0)),
        out_shape=jax.ShapeDtypeStruct((n_batch, n_hist, d), jnp.float32),
    )(g, w1d, b1d, w2d, b2d)


def _block_diag2(w):
    z = jnp.zeros_like(w)
    return jnp.block([[w, z], [z, w]])


def kernel(x, table, W1, b1, W2, b2):
    B, L = x.shape
    V, D = table.shape
    n_rows = B * L
    n_chunks = n_rows // (NW * CH)
    idx3 = (
        x.reshape(NW, n_chunks, CH // 2, 2)
        .transpose(0, 1, 3, 2)
        .reshape(NW, n_chunks, CH)
        .astype(jnp.int32)
    )
    gathered = _sc_gather(table, idx3, n_rows, D)
    w1d = _block_diag2(W1.T)
    w2d = _block_diag2(W2.T)
    b1d = jnp.concatenate([b1, b1]).reshape(1, 2 * D)
    b2d = jnp.concatenate([b2, b2]).reshape(1, 2 * D)
    return _tc_mlp(gathered, w1d, b1d, w2d, b2d, B, L, D)
